# baseline (speedup 1.0000x reference)
import jax
import jax.numpy as jnp
from jax import lax
from jax.experimental import pallas as pl
from jax.experimental.pallas import tpu as pltpu

BT = 2048
SEG = 1024
BD = 512


def _body(idx_ref, x_ref, w_ref, b_ref, a0_ref, a1_ref, bb0_ref, bb1_ref,
          o_ref, h_ref):
    xb = x_ref[...].astype(jnp.bfloat16)

    @pl.when(pl.program_id(1) == 0)
    def _():
        h0 = jnp.dot(xb[:SEG], a0_ref[0], preferred_element_type=jnp.float32)
        h1 = jnp.dot(xb[SEG:], a1_ref[0], preferred_element_type=jnp.float32)
        h_ref[:SEG] = h0.astype(jnp.bfloat16)
        h_ref[SEG:] = h1.astype(jnp.bfloat16)

    base = lax.dot_general(xb, w_ref[...], (((1,), (1,)), ((), ())),
                           preferred_element_type=jnp.float32)
    l0 = jnp.dot(h_ref[:SEG], bb0_ref[0], preferred_element_type=jnp.float32)
    l1 = jnp.dot(h_ref[SEG:], bb1_ref[0], preferred_element_type=jnp.float32)
    o_ref[:SEG, :] = b_ref[...] + base[:SEG] + l0
    o_ref[SEG:, :] = b_ref[...] + base[SEG:] + l1


def kernel(x, W, bias, lora_a, lora_b, indices):
    N, K = x.shape
    D = W.shape[0]
    L, _, R = lora_a.shape
    S = indices.shape[0] - 1
    nt = N // BT
    nd = D // BD

    w_bf = W.astype(jnp.bfloat16)
    a_bf = lora_a.astype(jnp.bfloat16)
    b_bf = lora_b.astype(jnp.bfloat16)
    bias2 = bias.reshape(1, D)

    def lora_of(seg_blk, idx_ref):
        seg = jnp.int32(0)
        for k in range(1, S):
            seg = seg + jnp.where(idx_ref[k, 0] <= seg_blk * SEG, 1, 0).astype(jnp.int32)
        return idx_ref[seg, 1]

    grid_spec = pltpu.PrefetchScalarGridSpec(
        num_scalar_prefetch=1,
        grid=(nt, nd),
        in_specs=[
            pl.BlockSpec((BT, K), lambda i, j, idx: (i, 0)),
            pl.BlockSpec((BD, K), lambda i, j, idx: (j, 0)),
            pl.BlockSpec((1, BD), lambda i, j, idx: (0, j)),
            pl.BlockSpec((1, K, R), lambda i, j, idx: (lora_of(2 * i, idx), 0, 0)),
            pl.BlockSpec((1, K, R), lambda i, j, idx: (lora_of(2 * i + 1, idx), 0, 0)),
            pl.BlockSpec((1, R, BD), lambda i, j, idx: (lora_of(2 * i, idx), 0, j)),
            pl.BlockSpec((1, R, BD), lambda i, j, idx: (lora_of(2 * i + 1, idx), 0, j)),
        ],
        out_specs=pl.BlockSpec((BT, BD), lambda i, j, idx: (i, j)),
        scratch_shapes=[pltpu.VMEM((BT, R), jnp.bfloat16)],
    )

    return pl.pallas_call(
        _body,
        grid_spec=grid_spec,
        out_shape=jax.ShapeDtypeStruct((N, D), x.dtype),
    )(indices, x, w_bf, bias2, a_bf, a_bf, b_bf, b_bf)
